# baseline (speedup 1.0000x reference)
import functools

import jax
import jax.numpy as jnp
from jax import lax
from jax.experimental import pallas as pl
from jax.experimental.pallas import tpu as pltpu
from jax.experimental.pallas import tpu_sc as plsc

B, C, H, W = 4, 32, 384, 512
NPB = H * W
N = B * NPB
D = 48
BLK = 4096
NBLK = N // BLK
NPAD = (NBLK + 1) * BLK
DUMMY = N

R = 32
RW = R * W
CPB = H // R
NCHUNK = B * CPB
NSUB = 16
SLAB = NPB // NSUB
RSTG = 2048
K = 128
TSH = RW // NSUB
OB = 128


def _prep_body(xt_ref, dep_ref, flw_ref, rows_ref, rfilt_ref):
    pid = pl.program_id(0)
    is_pad = pid >= NBLK
    i0 = pid * BLK
    gidx = i0 + lax.broadcasted_iota(jnp.int32, (BLK, 1), 0)
    rem = gidx % NPB
    hrow = (rem // W).astype(jnp.float32)
    wcol = (rem % W).astype(jnp.float32)

    fxv = jnp.clip(flw_ref[:, 0:1], -2.0 * W, 2.0 * W)
    fyv = jnp.clip(flw_ref[:, 1:2], -2.0 * W, 2.0 * W)
    xd = wcol + fxv
    yd = hrow + fyv
    x0 = jnp.floor(xd)
    y0 = jnp.floor(yd)
    fx = xd - x0
    fy = yd - y0
    c0 = x0.astype(jnp.int32)
    r0 = y0.astype(jnp.int32)

    lv = ((c0 >= 0) & (c0 <= W - 1)).astype(jnp.float32)
    rv = ((c0 + 1 >= 0) & (c0 + 1 <= W - 1)).astype(jnp.float32)
    wx0c = (1.0 - fx) * lv
    wx1c = fx * rv
    cc0 = jnp.clip(c0, 0, W - 1).astype(jnp.float32)
    cc1 = jnp.clip(c0 + 1, 0, W - 1).astype(jnp.float32)

    dwv = jnp.exp(-(jnp.clip(dep_ref[...], 0.001, 80.0) - 40.0) * (1.0 / 5.0))

    ones = jnp.ones_like(dwv)
    zeros7 = jnp.zeros((BLK, 7), jnp.float32)
    row = jnp.concatenate(
        [xt_ref[...] * dwv, dwv, ones, wx0c, wx1c, 1.0 - fy, fy, cc0, cc1,
         y0, zeros7], axis=1)
    rows_ref[...] = jnp.where(is_pad, jnp.zeros_like(row), row)
    rfilt_ref[...] = jnp.where(is_pad, jnp.full_like(r0, -(10 ** 6)), r0)


def _prep(xt, dep, flw):
    imap = lambda i: (jnp.minimum(i, NBLK - 1), 0)
    return pl.pallas_call(
        _prep_body,
        grid=(NBLK + 1,),
        in_specs=[
            pl.BlockSpec((BLK, C), imap),
            pl.BlockSpec((BLK, 1), imap),
            pl.BlockSpec((BLK, 2), imap),
        ],
        out_specs=[
            pl.BlockSpec((BLK, D), lambda i: (i, 0)),
            pl.BlockSpec((BLK, 1), lambda i: (i, 0)),
        ],
        out_shape=[
            jax.ShapeDtypeStruct((NPAD, D), jnp.float32),
            jax.ShapeDtypeStruct((NPAD, 1), jnp.int32),
        ],
    )(xt, dep, flw)


def _post_body(acc_ref, o_ref):
    dwf = acc_ref[:, 32:33]
    m = acc_ref[:, 33:34]
    scale = jnp.where(m >= 0.5, 1.0 / jnp.maximum(dwf, 1e-7), 0.0)
    o_ref[...] = acc_ref[:, 0:32] * scale


def _post(acc):
    return pl.pallas_call(
        _post_body,
        grid=(NBLK,),
        in_specs=[pl.BlockSpec((BLK, D), lambda i: (i, 0))],
        out_specs=pl.BlockSpec((BLK, C), lambda i: (i, 0)),
        out_shape=jax.ShapeDtypeStruct((N, C), jnp.float32),
    )(acc)


def _sc_body(rows_hbm, rfilt_hbm, out_hbm, acc, rbuf, lbuf, stage,
             wbuf, dbuf, zbuf, obuf, sem):
    cid = lax.axis_index("c")
    sid = lax.axis_index("s")

    pltpu.sync_copy(rows_hbm.at[pl.ds(N, OB)], zbuf)
    pltpu.sync_copy(rows_hbm.at[pl.ds(N, 4 * K)], wbuf)

    def _chunk(j, _):
        q = cid + 2 * j
        b = q // CPB
        lo = (q % CPB) * R

        for t in range(TSH // OB):
            pltpu.sync_copy(zbuf, acc.at[pl.ds(sid * TSH + t * OB, OB)])
        plsc.subcore_barrier()

        slab0 = b * NPB + sid * SLAB
        iota16 = lax.broadcasted_iota(jnp.int32, (16,), 0)

        def _piece(p, off):
            pltpu.sync_copy(rfilt_hbm.at[pl.ds(slab0 + p * RSTG, RSTG)], rbuf)

            def _scan(i, off):
                rv = rbuf[pl.ds(i * 16, 16)]
                m = (rv >= lo - 1) & (rv <= lo + R - 1)
                ids = slab0 + p * RSTG + i * 16 + iota16
                mi = m.astype(jnp.int32)
                pos = off + plsc.cumsum(mi) - mi
                plsc.store_scatter(lbuf, [pos], ids, mask=m)
                return off + jnp.sum(mi)
            return lax.fori_loop(0, RSTG // 16, _scan, off)
        off = lax.fori_loop(0, SLAB // RSTG, _piece, jnp.int32(0))

        for t in range(K // 16):
            lbuf[pl.ds(off + t * 16, 16)] = jnp.full((16,), DUMMY, jnp.int32)
        nst = (off + K - 1) // K

        def _stagef(g, _):
            pltpu.async_copy(rows_hbm.at[lbuf.at[pl.ds(g * K, K)]],
                             stage, sem).wait()

            for grp in range(K // 16):
                pix = lax.broadcasted_iota(jnp.int32, (16,), 0) + (grp * 16)

                def _meta(lane):
                    return plsc.load_gather(
                        stage, [pix, jnp.full((16,), lane, jnp.int32)])
                wx0c = _meta(34)
                wx1c = _meta(35)
                wy0 = _meta(36)
                wy1 = _meta(37)
                cc0 = _meta(38).astype(jnp.int32)
                cc1 = _meta(39).astype(jnp.int32)
                r0 = _meta(40).astype(jnp.int32)

                one = jnp.ones((16,), jnp.float32)
                zero = jnp.zeros((16,), jnp.float32)
                ft = jnp.where((r0 >= lo) & (r0 < lo + R), one, zero)
                fb = jnp.where((r0 + 1 >= lo) & (r0 + 1 < lo + R), one, zero)
                ws = (wx0c * wy0 * ft, wx1c * wy0 * ft,
                      wx0c * wy1 * fb, wx1c * wy1 * fb)
                rt = (r0 - lo) * W
                rb = rt + W
                ds_ = (jnp.clip(rt + cc0, 0, RW - 1),
                       jnp.clip(rt + cc1, 0, RW - 1),
                       jnp.clip(rb + cc0, 0, RW - 1),
                       jnp.clip(rb + cc1, 0, RW - 1))
                for t in range(4):
                    plsc.store_scatter(
                        dbuf, [jnp.full((16,), t, jnp.int32), pix], ds_[t])
                for ch in range(34):
                    chv = jnp.full((16,), ch, jnp.int32)
                    vec = plsc.load_gather(stage, [pix, chv])
                    for t in range(4):
                        plsc.store_scatter(
                            wbuf, [pix + (t * K), chv], vec * ws[t])

            for t in range(4):
                pltpu.sync_copy(wbuf.at[pl.ds(t * K, K)],
                                acc.at[dbuf.at[t]], add=True)
            return 0
        lax.fori_loop(0, nst, _stagef, 0)
        plsc.subcore_barrier()

        gbase = b * NPB + lo * W
        for t in range(TSH // OB):
            pltpu.sync_copy(acc.at[pl.ds(sid * TSH + t * OB, OB)], obuf)
            pltpu.sync_copy(obuf,
                            out_hbm.at[pl.ds(gbase + sid * TSH + t * OB, OB)])
        return 0

    lax.fori_loop(0, NCHUNK // 2, _chunk, 0)


def _sc_scatter(rows, rfilt):
    mesh = plsc.VectorSubcoreMesh(core_axis_name="c", subcore_axis_name="s")
    f = functools.partial(
        pl.kernel,
        out_type=jax.ShapeDtypeStruct((N, D), jnp.float32),
        mesh=mesh,
        compiler_params=pltpu.CompilerParams(use_tc_tiling_on_sc=False,
                                             needs_layout_passes=False),
        scratch_types=[
            pltpu.VMEM_SHARED((RW, D), jnp.float32),
            pltpu.VMEM((RSTG,), jnp.int32),
            pltpu.VMEM((SLAB + K,), jnp.int32),
            pltpu.VMEM((K, D), jnp.float32),
            pltpu.VMEM((4 * K, D), jnp.float32),
            pltpu.VMEM((4, K), jnp.int32),
            pltpu.VMEM((OB, D), jnp.float32),
            pltpu.VMEM((OB, D), jnp.float32),
            pltpu.SemaphoreType.DMA,
        ],
    )(_sc_body)
    return f(rows, rfilt)


def kernel(x, flow, depth):
    xt = jnp.transpose(x, (0, 2, 3, 1)).reshape(N, C)
    flw = jnp.transpose(flow, (0, 2, 3, 1)).reshape(N, 2)
    dep = jnp.transpose(depth, (0, 2, 3, 1)).reshape(N, 1)
    rows, rfilt = _prep(xt, dep, flw)
    rf = rfilt[:N].reshape(N)
    acc = _sc_scatter(rows, rf)
    y = _post(acc)
    return jnp.transpose(y.reshape(B, H, W, C), (0, 3, 1, 2))


# speedup vs baseline: 3.3419x; 3.3419x over previous
import functools

import jax
import jax.numpy as jnp
from jax import lax
from jax.experimental import pallas as pl
from jax.experimental.pallas import tpu as pltpu
from jax.experimental.pallas import tpu_sc as plsc

B, C, H, W = 4, 32, 384, 512
NPB = H * W
N = B * NPB
D = 48
BLK = 4096
NBLK = N // BLK
NPAD = (NBLK + 1) * BLK
DUMMY = N

R = 32
RW = R * W
CPB = H // R
NCHUNK = B * CPB
NSUB = 16
SLAB = NPB // NSUB
RSTG = 2048
K = 128
TSH = RW // NSUB
OB = 128


def _prep_body(xt_ref, dep_ref, flw_ref, rows_ref, rfilt_ref):
    pid = pl.program_id(0)
    is_pad = pid >= NBLK
    i0 = pid * BLK
    gidx = i0 + lax.broadcasted_iota(jnp.int32, (BLK, 1), 0)
    rem = gidx % NPB
    hrow = (rem // W).astype(jnp.float32)
    wcol = (rem % W).astype(jnp.float32)

    fxv = jnp.clip(flw_ref[:, 0:1], -2.0 * W, 2.0 * W)
    fyv = jnp.clip(flw_ref[:, 1:2], -2.0 * W, 2.0 * W)
    xd = wcol + fxv
    yd = hrow + fyv
    x0 = jnp.floor(xd)
    y0 = jnp.floor(yd)
    fx = xd - x0
    fy = yd - y0
    c0 = x0.astype(jnp.int32)
    r0 = y0.astype(jnp.int32)

    lv = ((c0 >= 0) & (c0 <= W - 1)).astype(jnp.float32)
    rv = ((c0 + 1 >= 0) & (c0 + 1 <= W - 1)).astype(jnp.float32)
    wx0c = (1.0 - fx) * lv
    wx1c = fx * rv
    cc0 = jnp.clip(c0, 0, W - 1).astype(jnp.float32)
    cc1 = jnp.clip(c0 + 1, 0, W - 1).astype(jnp.float32)

    dwv = jnp.exp(-(jnp.clip(dep_ref[...], 0.001, 80.0) - 40.0) * (1.0 / 5.0))

    ones = jnp.ones_like(dwv)
    zeros7 = jnp.zeros((BLK, 7), jnp.float32)
    row = jnp.concatenate(
        [xt_ref[...] * dwv, dwv, ones, wx0c, wx1c, 1.0 - fy, fy, cc0, cc1,
         y0, zeros7], axis=1)
    rows_ref[...] = jnp.where(is_pad, jnp.zeros_like(row), row)
    rfilt_ref[...] = jnp.where(is_pad, jnp.full_like(r0, -(10 ** 6)), r0)


def _prep(xt, dep, flw):
    imap = lambda i: (jnp.minimum(i, NBLK - 1), 0)
    return pl.pallas_call(
        _prep_body,
        grid=(NBLK + 1,),
        in_specs=[
            pl.BlockSpec((BLK, C), imap),
            pl.BlockSpec((BLK, 1), imap),
            pl.BlockSpec((BLK, 2), imap),
        ],
        out_specs=[
            pl.BlockSpec((BLK, D), lambda i: (i, 0)),
            pl.BlockSpec((BLK, 1), lambda i: (i, 0)),
        ],
        out_shape=[
            jax.ShapeDtypeStruct((NPAD, D), jnp.float32),
            jax.ShapeDtypeStruct((NPAD, 1), jnp.int32),
        ],
    )(xt, dep, flw)


def _post_body(acc_ref, o_ref):
    dwf = acc_ref[:, 32:33]
    m = acc_ref[:, 33:34]
    scale = jnp.where(m >= 0.5, 1.0 / jnp.maximum(dwf, 1e-7), 0.0)
    o_ref[...] = acc_ref[:, 0:32] * scale


def _post(acc):
    return pl.pallas_call(
        _post_body,
        grid=(NBLK,),
        in_specs=[pl.BlockSpec((BLK, D), lambda i: (i, 0))],
        out_specs=pl.BlockSpec((BLK, C), lambda i: (i, 0)),
        out_shape=jax.ShapeDtypeStruct((N, C), jnp.float32),
    )(acc)


def _sc_body(rows_hbm, rfilt_hbm, out_hbm, acc, rbuf, lbuf, stage,
             wbuf, dbuf, zbuf, obuf, sem):
    cid = lax.axis_index("c")
    sid = lax.axis_index("s")

    pltpu.sync_copy(rows_hbm.at[pl.ds(N, OB)], zbuf)
    pltpu.sync_copy(rows_hbm.at[pl.ds(N, 4 * K)], wbuf)

    def _chunk(j, _):
        q = cid + 2 * j
        b = q // CPB
        lo = (q % CPB) * R

        for t in range(TSH // OB):
            pltpu.sync_copy(zbuf, acc.at[pl.ds(sid * TSH + t * OB, OB)])
        plsc.subcore_barrier()

        iota16 = lax.broadcasted_iota(jnp.int32, (16,), 0)
        pltpu.sync_copy(rfilt_hbm.at[pl.ds((b * NSUB + sid) * SLAB, SLAB)],
                        rbuf)

        def _scan(i, off):
            p = i // (W // 16)
            w0 = (i % (W // 16)) * 16
            rv = rbuf[pl.ds(i * 16, 16)]
            m = (rv >= lo - 1) & (rv <= lo + R - 1)
            mi = m.astype(jnp.int32)
            s = jnp.sum(mi)

            @pl.when(s > 0)
            def _do():
                ids = b * NPB + (p * NSUB + sid) * W + w0 + iota16
                pos = off + plsc.cumsum(mi) - mi
                plsc.store_scatter(lbuf, [pos], ids, mask=m)
            return off + s
        off = lax.fori_loop(0, SLAB // 16, _scan, jnp.int32(0))

        for t in range(K // 16):
            lbuf[pl.ds(off + t * 16, 16)] = jnp.full((16,), DUMMY, jnp.int32)
        nst = (off + K - 1) // K

        def _stagef(g, _):
            pltpu.async_copy(rows_hbm.at[lbuf.at[pl.ds(g * K, K)]],
                             stage, sem).wait()

            for grp in range(K // 16):
                pix = lax.broadcasted_iota(jnp.int32, (16,), 0) + (grp * 16)

                def _meta(lane):
                    return plsc.load_gather(
                        stage, [pix, jnp.full((16,), lane, jnp.int32)])
                wx0c = _meta(34)
                wx1c = _meta(35)
                wy0 = _meta(36)
                wy1 = _meta(37)
                cc0 = _meta(38).astype(jnp.int32)
                cc1 = _meta(39).astype(jnp.int32)
                r0 = _meta(40).astype(jnp.int32)

                one = jnp.ones((16,), jnp.float32)
                zero = jnp.zeros((16,), jnp.float32)
                ft = jnp.where((r0 >= lo) & (r0 < lo + R), one, zero)
                fb = jnp.where((r0 + 1 >= lo) & (r0 + 1 < lo + R), one, zero)
                ws = (wx0c * wy0 * ft, wx1c * wy0 * ft,
                      wx0c * wy1 * fb, wx1c * wy1 * fb)
                rt = (r0 - lo) * W
                rb = rt + W
                ds_ = (jnp.clip(rt + cc0, 0, RW - 1),
                       jnp.clip(rt + cc1, 0, RW - 1),
                       jnp.clip(rb + cc0, 0, RW - 1),
                       jnp.clip(rb + cc1, 0, RW - 1))
                for t in range(4):
                    plsc.store_scatter(
                        dbuf, [jnp.full((16,), t, jnp.int32), pix], ds_[t])
                for ch in range(34):
                    chv = jnp.full((16,), ch, jnp.int32)
                    vec = plsc.load_gather(stage, [pix, chv])
                    for t in range(4):
                        plsc.store_scatter(
                            wbuf, [pix + (t * K), chv], vec * ws[t])

            for t in range(4):
                pltpu.sync_copy(wbuf.at[pl.ds(t * K, K)],
                                acc.at[dbuf.at[t]], add=True)
            return 0
        lax.fori_loop(0, nst, _stagef, 0)
        plsc.subcore_barrier()

        gbase = b * NPB + lo * W
        for t in range(TSH // OB):
            pltpu.sync_copy(acc.at[pl.ds(sid * TSH + t * OB, OB)], obuf)
            pltpu.sync_copy(obuf,
                            out_hbm.at[pl.ds(gbase + sid * TSH + t * OB, OB)])
        return 0

    lax.fori_loop(0, NCHUNK // 2, _chunk, 0)


def _sc_scatter(rows, rfilt):
    mesh = plsc.VectorSubcoreMesh(core_axis_name="c", subcore_axis_name="s")
    f = functools.partial(
        pl.kernel,
        out_type=jax.ShapeDtypeStruct((N, D), jnp.float32),
        mesh=mesh,
        compiler_params=pltpu.CompilerParams(use_tc_tiling_on_sc=False,
                                             needs_layout_passes=False),
        scratch_types=[
            pltpu.VMEM_SHARED((RW, D), jnp.float32),
            pltpu.VMEM((SLAB,), jnp.int32),
            pltpu.VMEM((SLAB + K,), jnp.int32),
            pltpu.VMEM((K, D), jnp.float32),
            pltpu.VMEM((4 * K, D), jnp.float32),
            pltpu.VMEM((4, K), jnp.int32),
            pltpu.VMEM((OB, D), jnp.float32),
            pltpu.VMEM((OB, D), jnp.float32),
            pltpu.SemaphoreType.DMA,
        ],
    )(_sc_body)
    return f(rows, rfilt)


def kernel(x, flow, depth):
    xt = jnp.transpose(x, (0, 2, 3, 1)).reshape(N, C)
    flw = jnp.transpose(flow, (0, 2, 3, 1)).reshape(N, 2)
    dep = jnp.transpose(depth, (0, 2, 3, 1)).reshape(N, 1)
    rows, rfilt = _prep(xt, dep, flw)
    rf = (rfilt[:N].reshape(B, H // 16, 16, W)
          .transpose(0, 2, 1, 3).reshape(N))
    acc = _sc_scatter(rows, rf)
    y = _post(acc)
    return jnp.transpose(y.reshape(B, H, W, C), (0, 3, 1, 2))


# speedup vs baseline: 6.3258x; 1.8929x over previous
import functools

import jax
import jax.numpy as jnp
from jax import lax
from jax.experimental import pallas as pl
from jax.experimental.pallas import tpu as pltpu
from jax.experimental.pallas import tpu_sc as plsc

B, C, H, W = 4, 32, 384, 512
NPB = H * W
N = B * NPB
D = 40
HB = 8
BLK = HB * W
JB = H // HB

R = 48
RW = R * W
CPB = H // R
NCHUNK = B * CPB
NSUB = 16
SLAB = NPB // NSUB
K = 128
TSH = RW // NSUB
OB = 128


def _prep_body(x_ref, flw_ref, dep_ref, rows_ref, rfp_ref):
    j = pl.program_id(1)
    hv = (j * HB + lax.broadcasted_iota(jnp.int32, (HB, W), 0)).astype(
        jnp.float32)
    wv = lax.broadcasted_iota(jnp.int32, (HB, W), 1).astype(jnp.float32)

    fxv = jnp.clip(flw_ref[0, 0], -2.0 * W, 2.0 * W)
    fyv = jnp.clip(flw_ref[0, 1], -2.0 * W, 2.0 * W)
    xd = wv + fxv
    yd = hv + fyv
    x0 = jnp.floor(xd)
    y0 = jnp.floor(yd)
    fx = xd - x0
    fy = yd - y0
    c0 = x0.astype(jnp.int32)
    r0 = y0.astype(jnp.int32)

    lv = ((c0 >= 0) & (c0 <= W - 1)).astype(jnp.float32)
    rv = ((c0 + 1 >= 0) & (c0 + 1 <= W - 1)).astype(jnp.float32)
    wx0c = (1.0 - fx) * lv
    wx1c = fx * rv
    ccp = (jnp.clip(c0, 0, W - 1)
           + W * jnp.clip(c0 + 1, 0, W - 1)).astype(jnp.float32)

    dw = jnp.exp(-(jnp.clip(dep_ref[0, 0], 0.001, 80.0) - 40.0) * 0.2)

    meta = jnp.stack([dw, wx0c, wx1c, 1.0 - fy, fy, ccp, y0, dw], axis=0)
    ones = jnp.ones((W, 1), jnp.float32)
    for r in range(HB):
        mt = jnp.transpose(meta[:, r, :])
        xt = jnp.transpose(x_ref[0, :, r, :])
        dwc = mt[:, 0:1]
        rows_ref[pl.ds(r * W, W), :] = jnp.concatenate(
            [xt * dwc, dwc, ones, mt[:, 1:7]], axis=1)
    rfp_ref[...] = r0[None, :, :]


def _prep(x, flow, depth):
    return pl.pallas_call(
        _prep_body,
        grid=(B, JB),
        in_specs=[
            pl.BlockSpec((1, C, HB, W), lambda b, j: (b, 0, j, 0)),
            pl.BlockSpec((1, 2, HB, W), lambda b, j: (b, 0, j, 0)),
            pl.BlockSpec((1, 1, HB, W), lambda b, j: (b, 0, j, 0)),
        ],
        out_specs=[
            pl.BlockSpec((BLK, D), lambda b, j: (b * JB + j, 0)),
            pl.BlockSpec((1, HB, W), lambda b, j: (b, j, 0)),
        ],
        out_shape=[
            jax.ShapeDtypeStruct((N, D), jnp.float32),
            jax.ShapeDtypeStruct((B, H, W), jnp.int32),
        ],
    )(x, flow, depth)


def _post_body(acc_ref, o_ref):
    a = acc_ref[...]
    scale = jnp.where(a[:, 33:34] >= 0.5,
                      1.0 / jnp.maximum(a[:, 32:33], 1e-7), 0.0)
    o = a[:, 0:32] * scale
    for r in range(HB):
        o_ref[0, :, r, :] = jnp.transpose(o[r * W:(r + 1) * W, :])


def _post(acc):
    return pl.pallas_call(
        _post_body,
        grid=(B, JB),
        in_specs=[pl.BlockSpec((BLK, D), lambda b, j: (b * JB + j, 0))],
        out_specs=pl.BlockSpec((1, C, HB, W), lambda b, j: (b, 0, j, 0)),
        out_shape=jax.ShapeDtypeStruct((B, C, H, W), jnp.float32),
    )(acc)


def _sc_body(rows_hbm, rfilt_hbm, zsrc_hbm, out_hbm, acc, rbuf, lbuf, stage,
             wbuf, dbuf, zbuf, obuf, sem):
    cid = lax.axis_index("c")
    sid = lax.axis_index("s")

    pltpu.sync_copy(zsrc_hbm, zbuf)
    for t in range(4):
        pltpu.sync_copy(zsrc_hbm, wbuf.at[pl.ds(t * K, K)])

    def _chunk(j, _):
        q = cid + 2 * j
        b = q // CPB
        lo = (q % CPB) * R

        for t in range(TSH // OB):
            pltpu.sync_copy(zbuf, acc.at[pl.ds(sid * TSH + t * OB, OB)])
        plsc.subcore_barrier()

        iota16 = lax.broadcasted_iota(jnp.int32, (16,), 0)
        pltpu.sync_copy(rfilt_hbm.at[pl.ds((b * NSUB + sid) * SLAB, SLAB)],
                        rbuf)

        def _scan(i, off):
            p = i // (W // 16)
            w0 = (i % (W // 16)) * 16
            rv = rbuf[pl.ds(i * 16, 16)]
            m = (rv >= lo - 1) & (rv <= lo + R - 1)
            mi = m.astype(jnp.int32)
            s = jnp.sum(mi)

            @pl.when(s > 0)
            def _do():
                ids = b * NPB + (p * NSUB + sid) * W + w0 + iota16
                pos = off + plsc.cumsum(mi) - mi
                plsc.store_scatter(lbuf, [pos], ids, mask=m)
            return off + s
        off = lax.fori_loop(0, SLAB // 16, _scan, jnp.int32(0))

        for t in range(K // 16):
            lbuf[pl.ds(off + t * 16, 16)] = jnp.zeros((16,), jnp.int32)
        nst = (off + K - 1) // K

        def _stagef(g, _):
            pltpu.async_copy(rows_hbm.at[lbuf.at[pl.ds(g * K, K)]],
                             stage, sem).wait()

            for grp in range(K // 16):
                pix = lax.broadcasted_iota(jnp.int32, (16,), 0) + (grp * 16)

                def _meta(lane):
                    return plsc.load_gather(
                        stage, [pix, jnp.full((16,), lane, jnp.int32)])
                wx0c = _meta(34)
                wx1c = _meta(35)
                wy0 = _meta(36)
                wy1 = _meta(37)
                ccp = _meta(38).astype(jnp.int32)
                r0 = _meta(39).astype(jnp.int32)
                cc0 = ccp % W
                cc1 = ccp // W

                one = jnp.ones((16,), jnp.float32)
                zero = jnp.zeros((16,), jnp.float32)
                ent = g * K + grp * 16 + iota16
                live = jnp.where(ent < off, one, zero)
                ft = jnp.where((r0 >= lo) & (r0 < lo + R), live, zero)
                fb = jnp.where((r0 + 1 >= lo) & (r0 + 1 < lo + R), live, zero)
                ws = (wx0c * wy0 * ft, wx1c * wy0 * ft,
                      wx0c * wy1 * fb, wx1c * wy1 * fb)
                rt = (r0 - lo) * W
                rb = rt + W
                ds_ = (jnp.clip(rt + cc0, 0, RW - 1),
                       jnp.clip(rt + cc1, 0, RW - 1),
                       jnp.clip(rb + cc0, 0, RW - 1),
                       jnp.clip(rb + cc1, 0, RW - 1))
                for t in range(4):
                    plsc.store_scatter(
                        dbuf, [jnp.full((16,), t, jnp.int32), pix], ds_[t])
                for ch in range(34):
                    chv = jnp.full((16,), ch, jnp.int32)
                    vec = plsc.load_gather(stage, [pix, chv])
                    for t in range(4):
                        plsc.store_scatter(
                            wbuf, [pix + (t * K), chv], vec * ws[t])

            for t in range(4):
                pltpu.sync_copy(wbuf.at[pl.ds(t * K, K)],
                                acc.at[dbuf.at[t]], add=True)
            return 0
        lax.fori_loop(0, nst, _stagef, 0)
        plsc.subcore_barrier()

        gbase = b * NPB + lo * W
        for t in range(TSH // OB):
            pltpu.sync_copy(acc.at[pl.ds(sid * TSH + t * OB, OB)], obuf)
            pltpu.sync_copy(obuf,
                            out_hbm.at[pl.ds(gbase + sid * TSH + t * OB, OB)])
        return 0

    lax.fori_loop(0, NCHUNK // 2, _chunk, 0)


def _sc_scatter(rows, rfilt, zsrc):
    mesh = plsc.VectorSubcoreMesh(core_axis_name="c", subcore_axis_name="s")
    f = functools.partial(
        pl.kernel,
        out_type=jax.ShapeDtypeStruct((N, D), jnp.float32),
        mesh=mesh,
        compiler_params=pltpu.CompilerParams(use_tc_tiling_on_sc=False,
                                             needs_layout_passes=False),
        scratch_types=[
            pltpu.VMEM_SHARED((RW, D), jnp.float32),
            pltpu.VMEM((SLAB,), jnp.int32),
            pltpu.VMEM((SLAB + K,), jnp.int32),
            pltpu.VMEM((K, D), jnp.float32),
            pltpu.VMEM((4 * K, D), jnp.float32),
            pltpu.VMEM((4, K), jnp.int32),
            pltpu.VMEM((OB, D), jnp.float32),
            pltpu.VMEM((OB, D), jnp.float32),
            pltpu.SemaphoreType.DMA,
        ],
    )(_sc_body)
    return f(rows, rfilt, zsrc)


def kernel(x, flow, depth):
    rows, rfp = _prep(x, flow, depth)
    rf = (rfp.reshape(B, H // NSUB, NSUB, W)
          .transpose(0, 2, 1, 3).reshape(N))
    zsrc = jnp.zeros((OB, D), jnp.float32)
    acc = _sc_scatter(rows, rf, zsrc)
    return _post(acc)


# speedup vs baseline: 6.9182x; 1.0937x over previous
import functools

import jax
import jax.numpy as jnp
from jax import lax
from jax.experimental import pallas as pl
from jax.experimental.pallas import tpu as pltpu
from jax.experimental.pallas import tpu_sc as plsc

B, C, H, W = 4, 32, 384, 512
NPB = H * W
N = B * NPB
D = 40
HB = 8
BLK = HB * W
JB = H // HB

R = 48
RW = R * W
CPB = H // R
NCHUNK = B * CPB
NSUB = 16
SLAB = NPB // NSUB
K = 128
TSH = RW // NSUB
OB = 128


def _prep_body(x_ref, flw_ref, dep_ref, rows_ref, rfp_ref):
    j = pl.program_id(1)
    hv = (j * HB + lax.broadcasted_iota(jnp.int32, (HB, W), 0)).astype(
        jnp.float32)
    wv = lax.broadcasted_iota(jnp.int32, (HB, W), 1).astype(jnp.float32)

    fxv = jnp.clip(flw_ref[0, 0], -2.0 * W, 2.0 * W)
    fyv = jnp.clip(flw_ref[0, 1], -2.0 * W, 2.0 * W)
    xd = wv + fxv
    yd = hv + fyv
    x0 = jnp.floor(xd)
    y0 = jnp.floor(yd)
    fx = xd - x0
    fy = yd - y0
    c0 = x0.astype(jnp.int32)
    r0 = y0.astype(jnp.int32)

    lv = ((c0 >= 0) & (c0 <= W - 1)).astype(jnp.float32)
    rv = ((c0 + 1 >= 0) & (c0 + 1 <= W - 1)).astype(jnp.float32)
    wx0c = (1.0 - fx) * lv
    wx1c = fx * rv
    ccp = (jnp.clip(c0, 0, W - 1)
           + W * jnp.clip(c0 + 1, 0, W - 1)).astype(jnp.float32)

    dw = jnp.exp(-(jnp.clip(dep_ref[0, 0], 0.001, 80.0) - 40.0) * 0.2)

    meta = jnp.stack([dw, wx0c, wx1c, 1.0 - fy, fy, ccp, y0, dw], axis=0)
    ones = jnp.ones((W, 1), jnp.float32)
    for r in range(HB):
        mt = jnp.transpose(meta[:, r, :])
        xt = jnp.transpose(x_ref[0, :, r, :])
        dwc = mt[:, 0:1]
        rows_ref[pl.ds(r * W, W), :] = jnp.concatenate(
            [xt * dwc, dwc, ones, mt[:, 1:7]], axis=1)
    rfp_ref[...] = r0[None, :, :]


def _prep(x, flow, depth):
    return pl.pallas_call(
        _prep_body,
        grid=(B, JB),
        in_specs=[
            pl.BlockSpec((1, C, HB, W), lambda b, j: (b, 0, j, 0)),
            pl.BlockSpec((1, 2, HB, W), lambda b, j: (b, 0, j, 0)),
            pl.BlockSpec((1, 1, HB, W), lambda b, j: (b, 0, j, 0)),
        ],
        out_specs=[
            pl.BlockSpec((BLK, D), lambda b, j: (b * JB + j, 0)),
            pl.BlockSpec((1, HB, W), lambda b, j: (b, j, 0)),
        ],
        out_shape=[
            jax.ShapeDtypeStruct((N, D), jnp.float32),
            jax.ShapeDtypeStruct((B, H, W), jnp.int32),
        ],
    )(x, flow, depth)


def _post_body(acc_ref, o_ref):
    a = acc_ref[...]
    scale = jnp.where(a[:, 33:34] >= 0.5,
                      1.0 / jnp.maximum(a[:, 32:33], 1e-7), 0.0)
    o = a[:, 0:32] * scale
    for r in range(HB):
        o_ref[0, :, r, :] = jnp.transpose(o[r * W:(r + 1) * W, :])


def _post(acc):
    return pl.pallas_call(
        _post_body,
        grid=(B, JB),
        in_specs=[pl.BlockSpec((BLK, D), lambda b, j: (b * JB + j, 0))],
        out_specs=pl.BlockSpec((1, C, HB, W), lambda b, j: (b, 0, j, 0)),
        out_shape=jax.ShapeDtypeStruct((B, C, H, W), jnp.float32),
    )(acc)


def _sc_body(rows_hbm, rfilt_hbm, zsrc_hbm, out_hbm, acc, rbuf, lbuf, stage,
             wbuf, dbuf, zbuf, obuf, sem_g, sem_s, sem_z, sem_o, sem_w):
    cid = lax.axis_index("c")
    sid = lax.axis_index("s")

    pltpu.sync_copy(zsrc_hbm, zbuf)
    for t in range(4):
        pltpu.sync_copy(zsrc_hbm, wbuf.at[pl.ds(t * K, K)])

    def _wait(sem, dst_shape_ref):
        pltpu.make_async_copy(zsrc_hbm, dst_shape_ref, sem).wait()

    def _chunk(j, _):
        q = cid + 2 * j
        b = q // CPB
        lo = (q % CPB) * R

        for t in range(TSH // OB):
            pltpu.async_copy(zbuf, acc.at[pl.ds(sid * TSH + t * OB, OB)],
                             sem_z)
        for t in range(TSH // OB):
            pltpu.make_async_copy(zbuf, acc.at[pl.ds(sid * TSH, OB)],
                                  sem_z).wait()
        plsc.subcore_barrier()

        iota16 = lax.broadcasted_iota(jnp.int32, (16,), 0)
        slab0 = (b * NSUB + sid) * SLAB

        def _half(ph, off):
            pltpu.sync_copy(
                rfilt_hbm.at[pl.ds(slab0 + ph * (SLAB // 2), SLAB // 2)],
                rbuf)

            def _scan(i, off):
                i2 = ph * (SLAB // 32) + i
                p = i2 // (W // 16)
                w0 = (i2 % (W // 16)) * 16
                rv = rbuf[pl.ds(i * 16, 16)]
                m = (rv >= lo - 1) & (rv <= lo + R - 1)
                mi = m.astype(jnp.int32)
                s = jnp.sum(mi)

                @pl.when(s > 0)
                def _do():
                    ids = b * NPB + (p * NSUB + sid) * W + w0 + iota16
                    pos = off + plsc.cumsum(mi) - mi
                    plsc.store_scatter(lbuf, [pos], ids, mask=m)
                return off + s
            return lax.fori_loop(0, SLAB // 32, _scan, off)
        off = lax.fori_loop(0, 2, _half, jnp.int32(0))

        for t in range(K // 16):
            lbuf[pl.ds(off + t * 16, 16)] = jnp.zeros((16,), jnp.int32)
        nst = (off + K - 1) // K

        @pl.when(nst > 0)
        def _pro():
            pltpu.async_copy(rows_hbm.at[lbuf.at[pl.ds(0, K)]],
                             stage.at[pl.ds(0, K)], sem_g)

        def _stagef(g, _):
            _wait(sem_g, stage.at[pl.ds(0, K)])

            @pl.when(g + 1 < nst)
            def _pf():
                pltpu.async_copy(
                    rows_hbm.at[lbuf.at[pl.ds((g + 1) * K, K)]],
                    stage.at[pl.ds(((g + 1) % 2) * K, K)], sem_g)

            @pl.when(g > 0)
            def _dr():
                for t in range(4):
                    _wait(sem_s, stage.at[pl.ds(0, K)])

            pixbase = (g % 2) * K

            for grp in range(K // 16):
                pixs = lax.broadcasted_iota(jnp.int32, (16,), 0) + (grp * 16)
                pix = pixs + pixbase

                def _meta(lane):
                    return plsc.load_gather(
                        stage, [pix, jnp.full((16,), lane, jnp.int32)])
                wx0c = _meta(34)
                wx1c = _meta(35)
                wy0 = _meta(36)
                wy1 = _meta(37)
                ccp = _meta(38).astype(jnp.int32)
                r0 = _meta(39).astype(jnp.int32)
                cc0 = ccp % W
                cc1 = ccp // W

                one = jnp.ones((16,), jnp.float32)
                zero = jnp.zeros((16,), jnp.float32)
                ent = g * K + grp * 16 + iota16
                live = jnp.where(ent < off, one, zero)
                ft = jnp.where((r0 >= lo) & (r0 < lo + R), live, zero)
                fb = jnp.where((r0 + 1 >= lo) & (r0 + 1 < lo + R), live, zero)
                ws = (wx0c * wy0 * ft, wx1c * wy0 * ft,
                      wx0c * wy1 * fb, wx1c * wy1 * fb)
                rt = (r0 - lo) * W
                rb = rt + W
                ds_ = (jnp.clip(rt + cc0, 0, RW - 1),
                       jnp.clip(rt + cc1, 0, RW - 1),
                       jnp.clip(rb + cc0, 0, RW - 1),
                       jnp.clip(rb + cc1, 0, RW - 1))
                for t in range(4):
                    plsc.store_scatter(
                        dbuf, [jnp.full((16,), t, jnp.int32), pixs], ds_[t])
                for ch in range(34):
                    chv = jnp.full((16,), ch, jnp.int32)
                    vec = plsc.load_gather(stage, [pix, chv])
                    for t in range(4):
                        plsc.store_scatter(
                            wbuf, [pixs + (t * K), chv], vec * ws[t])

            for t in range(4):
                pltpu.async_copy(wbuf.at[pl.ds(t * K, K)],
                                 acc.at[dbuf.at[t]], sem_s, add=True)
            return 0
        lax.fori_loop(0, nst, _stagef, 0)

        @pl.when(nst > 0)
        def _epi():
            for t in range(4):
                _wait(sem_s, stage.at[pl.ds(0, K)])
        plsc.subcore_barrier()

        gbase = b * NPB + lo * W
        NT = TSH // OB
        pltpu.async_copy(acc.at[pl.ds(sid * TSH, OB)],
                         obuf.at[pl.ds(0, OB)], sem_o)

        def _wb(t, _):
            half = (t % 2) * OB
            _wait(sem_o, obuf.at[pl.ds(0, OB)])

            @pl.when(t > 0)
            def _dw():
                _wait(sem_w, obuf.at[pl.ds(0, OB)])

            pltpu.async_copy(obuf.at[pl.ds(half, OB)],
                             out_hbm.at[pl.ds(gbase + sid * TSH + t * OB, OB)],
                             sem_w)

            @pl.when(t + 1 < NT)
            def _nr():
                pltpu.async_copy(
                    acc.at[pl.ds(sid * TSH + (t + 1) * OB, OB)],
                    obuf.at[pl.ds(((t + 1) % 2) * OB, OB)], sem_o)
            return 0
        lax.fori_loop(0, NT, _wb, 0)
        _wait(sem_w, obuf.at[pl.ds(0, OB)])
        return 0

    lax.fori_loop(0, NCHUNK // 2, _chunk, 0)


def _sc_scatter(rows, rfilt, zsrc):
    mesh = plsc.VectorSubcoreMesh(core_axis_name="c", subcore_axis_name="s")
    f = functools.partial(
        pl.kernel,
        out_type=jax.ShapeDtypeStruct((N, D), jnp.float32),
        mesh=mesh,
        compiler_params=pltpu.CompilerParams(use_tc_tiling_on_sc=False,
                                             needs_layout_passes=False),
        scratch_types=[
            pltpu.VMEM_SHARED((RW, D), jnp.float32),
            pltpu.VMEM((SLAB // 2,), jnp.int32),
            pltpu.VMEM((SLAB + K,), jnp.int32),
            pltpu.VMEM((2 * K, D), jnp.float32),
            pltpu.VMEM((4 * K, D), jnp.float32),
            pltpu.VMEM((4, K), jnp.int32),
            pltpu.VMEM((OB, D), jnp.float32),
            pltpu.VMEM((2 * OB, D), jnp.float32),
            pltpu.SemaphoreType.DMA,
            pltpu.SemaphoreType.DMA,
            pltpu.SemaphoreType.DMA,
            pltpu.SemaphoreType.DMA,
            pltpu.SemaphoreType.DMA,
        ],
    )(_sc_body)
    return f(rows, rfilt, zsrc)


def kernel(x, flow, depth):
    rows, rfp = _prep(x, flow, depth)
    rf = (rfp.reshape(B, H // NSUB, NSUB, W)
          .transpose(0, 2, 1, 3).reshape(N))
    zsrc = jnp.zeros((OB, D), jnp.float32)
    acc = _sc_scatter(rows, rf, zsrc)
    return _post(acc)


# speedup vs baseline: 7.0633x; 1.0210x over previous
import functools

import jax
import jax.numpy as jnp
from jax import lax
from jax.experimental import pallas as pl
from jax.experimental.pallas import tpu as pltpu
from jax.experimental.pallas import tpu_sc as plsc

B, C, H, W = 4, 32, 384, 512
NPB = H * W
N = B * NPB
D = 40
DW = 128
HB = 8
BLK = HB * W
JB = H // HB

R = 48
RW = R * W
CPB = H // R
NCHUNK = B * CPB
NSUB = 16
SLAB = NPB // NSUB
K = 64
TSH = RW // NSUB
OB = 128


def _prep_body(x_ref, flw_ref, dep_ref, rows_ref, rfp_ref):
    j = pl.program_id(1)
    hv = (j * HB + lax.broadcasted_iota(jnp.int32, (HB, W), 0)).astype(
        jnp.float32)
    wv = lax.broadcasted_iota(jnp.int32, (HB, W), 1).astype(jnp.float32)

    fxv = jnp.clip(flw_ref[0, 0], -2.0 * W, 2.0 * W)
    fyv = jnp.clip(flw_ref[0, 1], -2.0 * W, 2.0 * W)
    xd = wv + fxv
    yd = hv + fyv
    x0 = jnp.floor(xd)
    y0 = jnp.floor(yd)
    fx = xd - x0
    fy = yd - y0
    c0 = x0.astype(jnp.int32)
    r0 = y0.astype(jnp.int32)

    lv = ((c0 >= 0) & (c0 <= W - 1)).astype(jnp.float32)
    rv = ((c0 + 1 >= 0) & (c0 + 1 <= W - 1)).astype(jnp.float32)
    wx0c = (1.0 - fx) * lv
    wx1c = fx * rv
    ccp = (jnp.clip(c0, 0, W - 1)
           + W * jnp.clip(c0 + 1, 0, W - 1)).astype(jnp.float32)

    dw = jnp.exp(-(jnp.clip(dep_ref[0, 0], 0.001, 80.0) - 40.0) * 0.2)

    meta = jnp.stack([dw, wx0c, wx1c, 1.0 - fy, fy, ccp, y0, dw], axis=0)
    ones = jnp.ones((W, 1), jnp.float32)
    for r in range(HB):
        mt = jnp.transpose(meta[:, r, :])
        xt = jnp.transpose(x_ref[0, :, r, :])
        dwc = mt[:, 0:1]
        rows_ref[pl.ds(r * W, W), 0:D] = jnp.concatenate(
            [xt * dwc, dwc, ones, mt[:, 1:7]], axis=1)
    rfp_ref[...] = r0[None, :, :]


def _prep(x, flow, depth):
    return pl.pallas_call(
        _prep_body,
        grid=(B, JB),
        in_specs=[
            pl.BlockSpec((1, C, HB, W), lambda b, j: (b, 0, j, 0)),
            pl.BlockSpec((1, 2, HB, W), lambda b, j: (b, 0, j, 0)),
            pl.BlockSpec((1, 1, HB, W), lambda b, j: (b, 0, j, 0)),
        ],
        out_specs=[
            pl.BlockSpec((BLK, DW), lambda b, j: (b * JB + j, 0)),
            pl.BlockSpec((1, HB, W), lambda b, j: (b, j, 0)),
        ],
        out_shape=[
            jax.ShapeDtypeStruct((N, DW), jnp.float32),
            jax.ShapeDtypeStruct((B, H, W), jnp.int32),
        ],
    )(x, flow, depth)


def _post_body(acc_ref, o_ref):
    a = acc_ref[...]
    scale = jnp.where(a[:, 33:34] >= 0.5,
                      1.0 / jnp.maximum(a[:, 32:33], 1e-7), 0.0)
    o = a[:, 0:32] * scale
    for r in range(HB):
        o_ref[0, :, r, :] = jnp.transpose(o[r * W:(r + 1) * W, :])


def _post(acc):
    return pl.pallas_call(
        _post_body,
        grid=(B, JB),
        in_specs=[pl.BlockSpec((BLK, D), lambda b, j: (b * JB + j, 0))],
        out_specs=pl.BlockSpec((1, C, HB, W), lambda b, j: (b, 0, j, 0)),
        out_shape=jax.ShapeDtypeStruct((B, C, H, W), jnp.float32),
    )(acc)


def _sc_body(rows_hbm, rfilt_hbm, zsrc_hbm, out_hbm, acc, rbuf, lbuf, stage,
             wbuf, dbuf, zbuf, obuf, sem_g, sem_s, sem_z, sem_o, sem_w):
    cid = lax.axis_index("c")
    sid = lax.axis_index("s")

    pltpu.sync_copy(zsrc_hbm, zbuf)
    for t in range(4 * K // OB):
        pltpu.sync_copy(zsrc_hbm, wbuf.at[pl.ds(t * OB, OB)])

    def _wait_g(sem):
        pltpu.make_async_copy(rows_hbm.at[pl.ds(0, K)],
                              stage.at[pl.ds(0, K)], sem).wait()

    def _wait_s(sem):
        pltpu.make_async_copy(zsrc_hbm.at[pl.ds(0, K)],
                              acc.at[pl.ds(0, K)], sem).wait()

    def _wait_o(sem):
        pltpu.make_async_copy(zsrc_hbm, obuf.at[pl.ds(0, OB)], sem).wait()

    def _chunk(j, _):
        q = cid + 2 * j
        b = q // CPB
        lo = (q % CPB) * R

        for t in range(TSH // OB):
            pltpu.async_copy(zbuf, acc.at[pl.ds(sid * TSH + t * OB, OB)],
                             sem_z)
        for t in range(TSH // OB):
            pltpu.make_async_copy(zbuf, acc.at[pl.ds(sid * TSH, OB)],
                                  sem_z).wait()
        plsc.subcore_barrier()

        iota16 = lax.broadcasted_iota(jnp.int32, (16,), 0)
        slab0 = (b * NSUB + sid) * SLAB

        def _half(ph, off):
            pltpu.sync_copy(
                rfilt_hbm.at[pl.ds(slab0 + ph * (SLAB // 2), SLAB // 2)],
                rbuf)

            def _scan(i, off):
                i2 = ph * (SLAB // 32) + i
                p = i2 // (W // 16)
                w0 = (i2 % (W // 16)) * 16
                rv = rbuf[pl.ds(i * 16, 16)]
                m = (rv >= lo - 1) & (rv <= lo + R - 1)
                mi = m.astype(jnp.int32)
                s = jnp.sum(mi)

                @pl.when(s > 0)
                def _do():
                    ids = b * NPB + (p * NSUB + sid) * W + w0 + iota16
                    pos = off + plsc.cumsum(mi) - mi
                    plsc.store_scatter(lbuf, [pos], ids, mask=m)
                return off + s
            return lax.fori_loop(0, SLAB // 32, _scan, off)
        off = lax.fori_loop(0, 2, _half, jnp.int32(0))

        for t in range(K // 16):
            lbuf[pl.ds(off + t * 16, 16)] = jnp.zeros((16,), jnp.int32)
        nst = (off + K - 1) // K

        @pl.when(nst > 0)
        def _pro():
            pltpu.async_copy(rows_hbm.at[lbuf.at[pl.ds(0, K)]],
                             stage.at[pl.ds(0, K)], sem_g)

        def _stagef(g, _):
            _wait_g(sem_g)

            @pl.when(g + 1 < nst)
            def _pf():
                pltpu.async_copy(
                    rows_hbm.at[lbuf.at[pl.ds((g + 1) * K, K)]],
                    stage.at[pl.ds(((g + 1) % 2) * K, K)], sem_g)

            @pl.when(g > 0)
            def _dr():
                for t in range(4):
                    _wait_s(sem_s)

            pixbase = (g % 2) * K

            for grp in range(K // 16):
                pixs = lax.broadcasted_iota(jnp.int32, (16,), 0) + (grp * 16)
                pix = pixs + pixbase

                def _meta(lane):
                    return plsc.load_gather(
                        stage, [pix, jnp.full((16,), lane, jnp.int32)])
                wx0c = _meta(34)
                wx1c = _meta(35)
                wy0 = _meta(36)
                wy1 = _meta(37)
                ccp = _meta(38).astype(jnp.int32)
                r0 = _meta(39).astype(jnp.int32)
                cc0 = ccp % W
                cc1 = ccp // W

                one = jnp.ones((16,), jnp.float32)
                zero = jnp.zeros((16,), jnp.float32)
                ent = g * K + grp * 16 + iota16
                live = jnp.where(ent < off, one, zero)
                ft = jnp.where((r0 >= lo) & (r0 < lo + R), live, zero)
                fb = jnp.where((r0 + 1 >= lo) & (r0 + 1 < lo + R), live, zero)
                ws = (wx0c * wy0 * ft, wx1c * wy0 * ft,
                      wx0c * wy1 * fb, wx1c * wy1 * fb)
                rt = (r0 - lo) * W
                rb = rt + W
                ds_ = (jnp.clip(rt + cc0, 0, RW - 1),
                       jnp.clip(rt + cc1, 0, RW - 1),
                       jnp.clip(rb + cc0, 0, RW - 1),
                       jnp.clip(rb + cc1, 0, RW - 1))
                for t in range(4):
                    plsc.store_scatter(
                        dbuf, [jnp.full((16,), t, jnp.int32), pixs], ds_[t])
                for ch in range(34):
                    chv = jnp.full((16,), ch, jnp.int32)
                    vec = plsc.load_gather(stage, [pix, chv])
                    for t in range(4):
                        plsc.store_scatter(
                            wbuf, [pixs + (t * K), chv], vec * ws[t])

            for t in range(4):
                pltpu.async_copy(wbuf.at[pl.ds(t * K, K)],
                                 acc.at[dbuf.at[t]], sem_s, add=True)
            return 0
        lax.fori_loop(0, nst, _stagef, 0)

        @pl.when(nst > 0)
        def _epi():
            for t in range(4):
                _wait_s(sem_s)
        plsc.subcore_barrier()

        gbase = b * NPB + lo * W
        NT = TSH // OB
        pltpu.async_copy(acc.at[pl.ds(sid * TSH, OB)],
                         obuf.at[pl.ds(0, OB)], sem_o)

        def _wb(t, _):
            half = (t % 2) * OB
            _wait_o(sem_o)

            @pl.when(t > 0)
            def _dw():
                _wait_o(sem_w)

            pltpu.async_copy(obuf.at[pl.ds(half, OB)],
                             out_hbm.at[pl.ds(gbase + sid * TSH + t * OB, OB)],
                             sem_w)

            @pl.when(t + 1 < NT)
            def _nr():
                pltpu.async_copy(
                    acc.at[pl.ds(sid * TSH + (t + 1) * OB, OB)],
                    obuf.at[pl.ds(((t + 1) % 2) * OB, OB)], sem_o)
            return 0
        lax.fori_loop(0, NT, _wb, 0)
        _wait_o(sem_w)
        return 0

    lax.fori_loop(0, NCHUNK // 2, _chunk, 0)


def _sc_scatter(rows, rfilt, zsrc):
    mesh = plsc.VectorSubcoreMesh(core_axis_name="c", subcore_axis_name="s")
    f = functools.partial(
        pl.kernel,
        out_type=jax.ShapeDtypeStruct((N, D), jnp.float32),
        mesh=mesh,
        compiler_params=pltpu.CompilerParams(use_tc_tiling_on_sc=False,
                                             needs_layout_passes=False),
        scratch_types=[
            pltpu.VMEM_SHARED((RW, D), jnp.float32),
            pltpu.VMEM((SLAB // 2,), jnp.int32),
            pltpu.VMEM((SLAB + K,), jnp.int32),
            pltpu.VMEM((2 * K, DW), jnp.float32),
            pltpu.VMEM((4 * K, D), jnp.float32),
            pltpu.VMEM((4, K), jnp.int32),
            pltpu.VMEM((OB, D), jnp.float32),
            pltpu.VMEM((2 * OB, D), jnp.float32),
            pltpu.SemaphoreType.DMA,
            pltpu.SemaphoreType.DMA,
            pltpu.SemaphoreType.DMA,
            pltpu.SemaphoreType.DMA,
            pltpu.SemaphoreType.DMA,
        ],
    )(_sc_body)
    return f(rows, rfilt, zsrc)


def kernel(x, flow, depth):
    rows, rfp = _prep(x, flow, depth)
    rf = (rfp.reshape(B, H // NSUB, NSUB, W)
          .transpose(0, 2, 1, 3).reshape(N))
    zsrc = jnp.zeros((OB, D), jnp.float32)
    acc = _sc_scatter(rows, rf, zsrc)
    return _post(acc)
